# baseline (speedup 1.0000x reference)
import functools

import jax
import jax.numpy as jnp
from jax import lax
from jax.experimental import pallas as pl
from jax.experimental.pallas import tpu as pltpu
from jax.experimental.pallas import tpu_sc as plsc

VOCAB = 14762
EMB = 300
SEQ = 50
TGT = 2
BATCH = 16384

NW = 32
S_PER_W = BATCH // NW
IDX_MINOR = 128
XF_PER_J = S_PER_W // IDX_MINOR


def _mm_body(tb_ref, wf_ref, bcol_ref, x_ref, v_ref, xt_ref):
    t = jnp.maximum(tb_ref[...], jnp.bfloat16(0))
    acc = lax.dot_general(wf_ref[...], t, (((1,), (1,)), ((), ())),
                          preferred_element_type=jnp.float32)
    acc = acc + bcol_ref[...]
    u0 = lax.bitcast_convert_type(
        acc[:SEQ].astype(jnp.bfloat16), jnp.uint16).astype(jnp.uint32)
    u1 = lax.bitcast_convert_type(
        acc[SEQ:].astype(jnp.bfloat16), jnp.uint16).astype(jnp.uint32)
    v_ref[...] = lax.bitcast_convert_type(u0 | (u1 << 16), jnp.int32)
    xt_ref[...] = x_ref[...].T


def _stage_a(table, wfold, bcol, x):
    vc = 4096
    grid = (BATCH + vc - 1) // vc
    return pl.pallas_call(
        _mm_body,
        grid=(grid,),
        in_specs=[
            pl.BlockSpec((vc, EMB), lambda i: (i, 0)),
            pl.BlockSpec((SEQ * TGT, EMB), lambda i: (0, 0)),
            pl.BlockSpec((SEQ * TGT, 1), lambda i: (0, 0)),
            pl.BlockSpec((vc, SEQ), lambda i: (i, 0)),
        ],
        out_specs=[
            pl.BlockSpec((SEQ, vc), lambda i: (0, i)),
            pl.BlockSpec((SEQ, vc), lambda i: (0, i)),
        ],
        out_shape=[
            jax.ShapeDtypeStruct((SEQ, VOCAB), jnp.int32),
            jax.ShapeDtypeStruct((SEQ, BATCH), jnp.int32),
        ],
    )(table, wfold, bcol, x)


BOUNCE = 5768
STAGE_CHUNK = 8 * BOUNCE
VSH_SIZE = 16 * STAGE_CHUNK


def _sc_body(v2_hbm, xt_hbm, o0_hbm, o1_hbm,
             xbuf, idxa, dst, ob0, ob1, vtmp, vsh, semg, semx):
    wid = lax.axis_index("s") * 2 + lax.axis_index("c")
    sid = lax.axis_index("s")
    base = wid * S_PER_W

    xcp = pltpu.make_async_copy(
        xt_hbm.at[:, pl.ds(base, S_PER_W)], xbuf, semx)
    xcp.start()

    def bounce(off, size):
        pltpu.sync_copy(v2_hbm.at[pl.ds(off, size)], vtmp.at[pl.ds(0, size)])
        pltpu.sync_copy(vtmp.at[pl.ds(0, size)], vsh.at[pl.ds(off, size)])

    n_full = jnp.where(sid == 15, 7, 8)

    def stage_body(k, _):
        bounce(sid * STAGE_CHUNK + k * BOUNCE, BOUNCE)
        return 0

    lax.fori_loop(0, n_full, stage_body, 0)

    @pl.when(sid == 15)
    def _():
        off = 15 * STAGE_CHUNK + 7 * BOUNCE
        bounce(off, SEQ * VOCAB - 15 * STAGE_CHUNK - 7 * BOUNCE)

    plsc.subcore_barrier()
    xcp.wait()

    def gxfer_args(j, r):
        return (vsh.at[idxa.at[lax.rem(j, 2) * XF_PER_J + r]],
                dst.at[lax.rem(j, 2)].at[pl.ds(r * IDX_MINOR, IDX_MINOR)],
                semg)

    himask = jnp.full((16,), -65536, jnp.int32)

    def accum(j, first):
        p = lax.rem(j, 2)
        for c in range(S_PER_W // 16):
            w = dst[p, pl.ds(c * 16, 16)]
            t0 = lax.bitcast_convert_type(w << 16, jnp.float32)
            t1 = lax.bitcast_convert_type(w & himask, jnp.float32)
            if first:
                ob0[pl.ds(c * 16, 16)] = t0
                ob1[pl.ds(c * 16, 16)] = t1
            else:
                ob0[pl.ds(c * 16, 16)] = ob0[pl.ds(c * 16, 16)] + t0
                ob1[pl.ds(c * 16, 16)] = ob1[pl.ds(c * 16, 16)] + t1

    def pipe_body(j, _):
        jbase = j * VOCAB
        for c in range(S_PER_W // 16):
            a = xbuf[j, pl.ds(c * 16, 16)] + jbase
            idxa[lax.rem(j, 2) * XF_PER_J + c // 8,
                 pl.ds((c % 8) * 16, 16)] = a

        for r in range(XF_PER_J):
            pltpu.async_copy(*gxfer_args(j, r))

        @pl.when(j > 0)
        def _():
            for r in range(XF_PER_J):
                pltpu.make_async_copy(*gxfer_args(j - 1, r)).wait()

        @pl.when(j == 1)
        def _():
            accum(0, True)

        @pl.when(j > 1)
        def _():
            accum(j - 1, False)

        return 0

    lax.fori_loop(0, SEQ, pipe_body, 0)
    for r in range(XF_PER_J):
        pltpu.make_async_copy(*gxfer_args(SEQ - 1, r)).wait()
    accum(SEQ - 1, False)

    pltpu.sync_copy(ob0, o0_hbm.at[pl.ds(base, S_PER_W)])
    pltpu.sync_copy(ob1, o1_hbm.at[pl.ds(base, S_PER_W)])


def _stage_b(v2, xt):
    mesh = plsc.VectorSubcoreMesh(core_axis_name="c", subcore_axis_name="s")
    fn = functools.partial(
        pl.kernel,
        mesh=mesh,
        out_type=[jax.ShapeDtypeStruct((BATCH,), jnp.float32),
                  jax.ShapeDtypeStruct((BATCH,), jnp.float32)],
        scratch_types=[
            pltpu.VMEM((SEQ, S_PER_W), jnp.int32),
            pltpu.VMEM((2 * XF_PER_J, IDX_MINOR), jnp.int32),
            pltpu.VMEM((2, S_PER_W), jnp.int32),
            pltpu.VMEM((S_PER_W,), jnp.float32),
            pltpu.VMEM((S_PER_W,), jnp.float32),
            pltpu.VMEM((BOUNCE,), jnp.int32),
            pltpu.VMEM_SHARED((VSH_SIZE,), jnp.int32),
            pltpu.SemaphoreType.DMA,
            pltpu.SemaphoreType.DMA,
        ],
    )(_sc_body)
    return fn(v2, xt)


def kernel(x, table, W, b):
    wfold = W.reshape(SEQ * TGT, EMB).astype(jnp.bfloat16)
    bcol = jnp.zeros((SEQ * TGT, 1), jnp.float32)
    bcol = bcol.at[0, 0].set(b[0]).at[SEQ, 0].set(b[1])

    v2_2d, xt = _stage_a(table.astype(jnp.bfloat16), wfold, bcol,
                         x.astype(jnp.int32))
    o0, o1 = _stage_b(v2_2d.reshape(-1), xt)
    return jnp.stack([o0, o1], axis=1)


# speedup vs baseline: 1.0164x; 1.0164x over previous
import functools

import jax
import jax.numpy as jnp
from jax import lax
from jax.experimental import pallas as pl
from jax.experimental.pallas import tpu as pltpu
from jax.experimental.pallas import tpu_sc as plsc

VOCAB = 14762
EMB = 300
SEQ = 50
TGT = 2
BATCH = 16384

NW = 32
S_PER_W = BATCH // NW
IDX_MINOR = 128
XF_PER_J = S_PER_W // IDX_MINOR


def _mm_body(tb_ref, wf_ref, bcol_ref, x_ref, v_ref, xt_ref):
    t = jnp.maximum(tb_ref[...], jnp.bfloat16(0))
    acc = lax.dot_general(wf_ref[...], t, (((1,), (1,)), ((), ())),
                          preferred_element_type=jnp.float32)
    acc = acc + bcol_ref[...]
    u0 = lax.bitcast_convert_type(
        acc[:SEQ].astype(jnp.bfloat16), jnp.uint16).astype(jnp.uint32)
    u1 = lax.bitcast_convert_type(
        acc[SEQ:].astype(jnp.bfloat16), jnp.uint16).astype(jnp.uint32)
    v_ref[...] = lax.bitcast_convert_type(u0 | (u1 << 16), jnp.int32)
    xt_ref[...] = x_ref[...].T


def _stage_a(table, wfold, bcol, x):
    vc = 4096
    grid = (BATCH + vc - 1) // vc
    return pl.pallas_call(
        _mm_body,
        grid=(grid,),
        in_specs=[
            pl.BlockSpec((vc, EMB), lambda i: (i, 0)),
            pl.BlockSpec((SEQ * TGT, EMB), lambda i: (0, 0)),
            pl.BlockSpec((SEQ * TGT, 1), lambda i: (0, 0)),
            pl.BlockSpec((vc, SEQ), lambda i: (i, 0)),
        ],
        out_specs=[
            pl.BlockSpec((SEQ, vc), lambda i: (0, i)),
            pl.BlockSpec((SEQ, vc), lambda i: (0, i)),
        ],
        out_shape=[
            jax.ShapeDtypeStruct((SEQ, VOCAB), jnp.int32),
            jax.ShapeDtypeStruct((SEQ, BATCH), jnp.int32),
        ],
    )(table, wfold, bcol, x)


BOUNCE = 5768
STAGE_CHUNK = 8 * BOUNCE
VSH_SIZE = 16 * STAGE_CHUNK


def _sc_body(v2_hbm, xt_hbm, o0_hbm, o1_hbm,
             xbuf, idxa, dst, ob0, ob1, vtmp, vsh, semg, semx):
    wid = lax.axis_index("s") * 2 + lax.axis_index("c")
    sid = lax.axis_index("s")
    base = wid * S_PER_W

    xcp = pltpu.make_async_copy(
        xt_hbm.at[:, pl.ds(base, S_PER_W)], xbuf, semx)
    xcp.start()

    def bounce(off, size):
        pltpu.sync_copy(v2_hbm.at[pl.ds(off, size)], vtmp.at[pl.ds(0, size)])
        pltpu.sync_copy(vtmp.at[pl.ds(0, size)], vsh.at[pl.ds(off, size)])

    n_full = jnp.where(sid == 15, 7, 8)

    def stage_body(k, _):
        bounce(sid * STAGE_CHUNK + k * BOUNCE, BOUNCE)
        return 0

    lax.fori_loop(0, n_full, stage_body, 0)

    @pl.when(sid == 15)
    def _():
        off = 15 * STAGE_CHUNK + 7 * BOUNCE
        bounce(off, SEQ * VOCAB - 15 * STAGE_CHUNK - 7 * BOUNCE)

    plsc.subcore_barrier()
    xcp.wait()

    def gxfer_args(j, r):
        return (vsh.at[idxa.at[lax.rem(j, 2) * XF_PER_J + r]],
                dst.at[j].at[pl.ds(r * IDX_MINOR, IDX_MINOR)],
                semg)

    def pipe_body(j, _):
        jbase = j * VOCAB
        for c in range(S_PER_W // 16):
            a = xbuf[j, pl.ds(c * 16, 16)] + jbase
            idxa[lax.rem(j, 2) * XF_PER_J + c // 8,
                 pl.ds((c % 8) * 16, 16)] = a

        for r in range(XF_PER_J):
            pltpu.async_copy(*gxfer_args(j, r))

        @pl.when(j > 0)
        def _():
            for r in range(XF_PER_J):
                pltpu.make_async_copy(*gxfer_args(j - 1, r)).wait()

        return 0

    lax.fori_loop(0, SEQ, pipe_body, 0)
    for r in range(XF_PER_J):
        pltpu.make_async_copy(*gxfer_args(SEQ - 1, r)).wait()

    zeros = jnp.zeros((16,), jnp.float32)
    himask = jnp.full((16,), -65536, jnp.int32)

    def seg_body(gi, _):
        i0 = gi * 16

        def j_body(j, accs):
            a0, a1 = accs
            w = dst[j, pl.ds(i0, 16)]
            t0 = lax.bitcast_convert_type(w << 16, jnp.float32)
            t1 = lax.bitcast_convert_type(w & himask, jnp.float32)
            return a0 + t0, a1 + t1

        acc0, acc1 = lax.fori_loop(0, SEQ, j_body, (zeros, zeros), unroll=5)
        ob0[pl.ds(i0, 16)] = acc0
        ob1[pl.ds(i0, 16)] = acc1
        return 0

    lax.fori_loop(0, S_PER_W // 16, seg_body, 0)

    pltpu.sync_copy(ob0, o0_hbm.at[pl.ds(base, S_PER_W)])
    pltpu.sync_copy(ob1, o1_hbm.at[pl.ds(base, S_PER_W)])


def _stage_b(v2, xt):
    mesh = plsc.VectorSubcoreMesh(core_axis_name="c", subcore_axis_name="s")
    fn = functools.partial(
        pl.kernel,
        mesh=mesh,
        out_type=[jax.ShapeDtypeStruct((BATCH,), jnp.float32),
                  jax.ShapeDtypeStruct((BATCH,), jnp.float32)],
        scratch_types=[
            pltpu.VMEM((SEQ, S_PER_W), jnp.int32),
            pltpu.VMEM((2 * XF_PER_J, IDX_MINOR), jnp.int32),
            pltpu.VMEM((SEQ, S_PER_W), jnp.int32),
            pltpu.VMEM((S_PER_W,), jnp.float32),
            pltpu.VMEM((S_PER_W,), jnp.float32),
            pltpu.VMEM((BOUNCE,), jnp.int32),
            pltpu.VMEM_SHARED((VSH_SIZE,), jnp.int32),
            pltpu.SemaphoreType.DMA,
            pltpu.SemaphoreType.DMA,
        ],
    )(_sc_body)
    return fn(v2, xt)


def kernel(x, table, W, b):
    wfold = W.reshape(SEQ * TGT, EMB).astype(jnp.bfloat16)
    bcol = jnp.zeros((SEQ * TGT, 1), jnp.float32)
    bcol = bcol.at[0, 0].set(b[0]).at[SEQ, 0].set(b[1])

    v2_2d, xt = _stage_a(table.astype(jnp.bfloat16), wfold, bcol,
                         x.astype(jnp.int32))
    o0, o1 = _stage_b(v2_2d.reshape(-1), xt)
    return jnp.stack([o0, o1], axis=1)
